# baseline (speedup 1.0000x reference)
import functools

import jax
import jax.numpy as jnp
from jax import lax
from jax.experimental import pallas as pl
from jax.experimental.pallas import tpu as pltpu
from jax.experimental.pallas import tpu_sc as plsc

NC = 2
NS = 16
NW = NC * NS
LANES = 16

C_BINS = 272
PLANES = 4
C_PACK = C_BINS // PLANES
C_PAD = 80


def _build_hist(b: int, n: int, n_idx: int):
    n_rows = b * n
    assert b % NW == 0
    b_per_w = b // NW
    rows_per_w = b_per_w * n
    assert b_per_w % LANES == 0
    b_groups = b_per_w // LANES
    SLAB = 128
    w_per_slab = SLAB // b_per_w
    i_chunk = n // w_per_slab
    mesh = plsc.VectorSubcoreMesh(core_axis_name="c", subcore_axis_name="s")

    rows_per_half = rows_per_w // b_groups
    n_ii = n // i_chunk
    n_stages = b_groups * n_ii

    @functools.partial(
        pl.kernel,
        out_type=jax.ShapeDtypeStruct((n_rows, C_PAD), jnp.int32),
        mesh=mesh,
        compiler_params=pltpu.CompilerParams(needs_layout_passes=False),
        scratch_types=[
            pltpu.VMEM((2, i_chunk, n_idx, SLAB), jnp.int32),
            pltpu.VMEM((rows_per_half, C_PAD), jnp.int32),
            pltpu.SemaphoreType.DMA,
            pltpu.SemaphoreType.DMA,
            pltpu.SemaphoreType.DMA,
        ],
    )
    def hist(idx_hbm, cnt_hbm, idx_v, cnt_v, sem0, sem1, out_sem):
        wid = lax.axis_index("s") * NC + lax.axis_index("c")
        slab = wid // w_per_slab
        quarter = wid % w_per_slab
        zeros = jnp.zeros((LANES,), jnp.int32)
        row_step = lax.iota(jnp.int32, LANES) * n
        sems = (sem0, sem1)

        def start(s):
            ii = s % n_ii
            return pltpu.async_copy(
                idx_hbm.at[
                    pl.ds(ii * i_chunk, i_chunk), :, pl.ds(slab * SLAB, SLAB)
                ],
                idx_v.at[s % 2],
                sems[s % 2],
            )

        pending = {0: start(0)}
        out_pending = None
        for g in range(b_groups):
            @plsc.parallel_loop(0, rows_per_half, unroll=8)
            def _zero(r):
                for c in range(C_PAD // LANES):
                    cnt_v[r, pl.ds(c * LANES, LANES)] = zeros

            for ii in range(n_ii):
                s = g * n_ii + ii
                if s + 1 < n_stages:
                    pending[s + 1] = start(s + 1)
                pending.pop(s).wait()
                buf = s % 2

                @plsc.parallel_loop(0, i_chunk * 4, unroll=1)
                def _scatter(t):
                    il = t // 4
                    jg = t % 4
                    rows = row_step + (ii * i_chunk + il)
                    lane0 = quarter * b_per_w + g * LANES
                    for jj in range(n_idx // 4):
                        j = jg * (n_idx // 4) + jj
                        v = idx_v[buf, il, j, pl.ds(lane0, LANES)]
                        q = (
                            (v >= C_PACK).astype(jnp.int32)
                            + (v >= 2 * C_PACK).astype(jnp.int32)
                            + (v >= 3 * C_PACK).astype(jnp.int32)
                        )
                        col = v - q * C_PACK
                        val = lax.shift_left(
                            jnp.full((LANES,), 1, jnp.int32), q * 8
                        )
                        plsc.addupdate_scatter(cnt_v, [rows, col], val)

            out_pending = pltpu.async_copy(
                cnt_v,
                cnt_hbm.at[
                    pl.ds(wid * rows_per_w + g * rows_per_half, rows_per_half)
                ],
                out_sem,
            )
            if g + 1 < b_groups:
                out_pending.wait()
        out_pending.wait()

    return hist


def _mm_body(cnt_ref, cb_ref, o_ref):
    packed = cnt_ref[...]
    acc = None
    for p in range(PLANES):
        plane = lax.shift_right_logical(packed, 8 * p) & 0xFF
        term = jnp.dot(
            plane.astype(jnp.bfloat16),
            cb_ref[p],
            preferred_element_type=jnp.float32,
        )
        acc = term if acc is None else acc + term
    o_ref[...] = acc


def _build_matmul(n_rows: int, d: int, block_rows: int):
    grid = (n_rows // block_rows,)
    return pl.pallas_call(
        _mm_body,
        grid=grid,
        in_specs=[
            pl.BlockSpec((block_rows, C_PAD), lambda i: (i, 0)),
            pl.BlockSpec((PLANES, C_PAD, d), lambda i: (0, 0, 0)),
        ],
        out_specs=pl.BlockSpec((block_rows, d), lambda i: (i, 0)),
        out_shape=jax.ShapeDtypeStruct((n_rows, d), jnp.float32),
    )


@functools.lru_cache(maxsize=None)
def _build(b, n, j, v, d):
    n_rows = b * n
    hist = _build_hist(b, n, j)
    matmul = _build_matmul(n_rows, d, block_rows=1024)

    def run(idx, codebook):
        idx_t = jnp.transpose(idx.astype(jnp.int32), (1, 2, 0))
        counts = hist(idx_t)
        cb_pad = jnp.pad(codebook.astype(jnp.float32), ((0, C_BINS - v), (0, 0)))
        cb_planes = jnp.pad(
            cb_pad.reshape(PLANES, C_PACK, d), ((0, 0), (0, C_PAD - C_PACK), (0, 0))
        ).astype(jnp.bfloat16)
        return matmul(counts, cb_planes).reshape(b, n, d)

    return run


def kernel(node2node_shortest_path_length_type, codebook):
    b, n, j = node2node_shortest_path_length_type.shape
    v, d = codebook.shape
    return _build(b, n, j, v, d)(node2node_shortest_path_length_type, codebook)


# speedup vs baseline: 1.1600x; 1.1600x over previous
import functools

import jax
import jax.numpy as jnp
from jax import lax
from jax.experimental import pallas as pl
from jax.experimental.pallas import tpu as pltpu
from jax.experimental.pallas import tpu_sc as plsc

NC = 2
NS = 16
NW = NC * NS
LANES = 16

C_BINS = 272
PLANES = 4
C_PACK = C_BINS // PLANES
C_PAD = 80


def _build_hist(b: int, n: int, n_idx: int):
    n_rows = b * n
    assert b % NW == 0
    b_per_w = b // NW
    rows_per_w = b_per_w * n
    assert b_per_w % LANES == 0
    b_groups = b_per_w // LANES
    SLAB = 128
    w_per_slab = SLAB // b_per_w
    i_chunk = n // w_per_slab
    mesh = plsc.VectorSubcoreMesh(core_axis_name="c", subcore_axis_name="s")

    rows_per_half = rows_per_w // b_groups
    n_ii = n // i_chunk
    n_stages = b_groups * n_ii

    @functools.partial(
        pl.kernel,
        out_type=jax.ShapeDtypeStruct((n_rows, C_PAD), jnp.int32),
        mesh=mesh,
        compiler_params=pltpu.CompilerParams(needs_layout_passes=False),
        scratch_types=[
            pltpu.VMEM((2, i_chunk, n_idx, SLAB), jnp.int32),
            pltpu.VMEM((rows_per_half, C_PAD), jnp.int32),
            pltpu.SemaphoreType.DMA,
            pltpu.SemaphoreType.DMA,
            pltpu.SemaphoreType.DMA,
        ],
    )
    def hist(idx_hbm, cnt_hbm, idx_v, cnt_v, sem0, sem1, out_sem):
        wid = lax.axis_index("s") * NC + lax.axis_index("c")
        slab = wid // w_per_slab
        quarter = wid % w_per_slab
        zeros = jnp.zeros((LANES,), jnp.int32)
        row_step = lax.iota(jnp.int32, LANES) * n
        sems = (sem0, sem1)

        def start(s):
            ii = s % n_ii
            return pltpu.async_copy(
                idx_hbm.at[
                    pl.ds(ii * i_chunk, i_chunk), :, pl.ds(slab * SLAB, SLAB)
                ],
                idx_v.at[s % 2],
                sems[s % 2],
            )

        pending = {0: start(0)}
        out_pending = None
        for g in range(b_groups):
            @plsc.parallel_loop(0, rows_per_half, unroll=8)
            def _zero(r):
                for c in range(C_PAD // LANES):
                    cnt_v[r, pl.ds(c * LANES, LANES)] = zeros

            for ii in range(n_ii):
                s = g * n_ii + ii
                if s + 1 < n_stages:
                    pending[s + 1] = start(s + 1)
                pending.pop(s).wait()
                buf = s % 2

                @plsc.parallel_loop(0, i_chunk * 4, unroll=1)
                def _scatter(t):
                    il = t // 4
                    jg = t % 4
                    rows = row_step + (ii * i_chunk + il)
                    lane0 = quarter * b_per_w + g * LANES
                    for jj in range(n_idx // 4):
                        j = jg * (n_idx // 4) + jj
                        v = idx_v[buf, il, j, pl.ds(lane0, LANES)]
                        q = (
                            (v >= C_PACK).astype(jnp.int32)
                            + (v >= 2 * C_PACK).astype(jnp.int32)
                            + (v >= 3 * C_PACK).astype(jnp.int32)
                        )
                        col = v - q * C_PACK
                        val = lax.shift_left(
                            jnp.full((LANES,), 1, jnp.int32), q * 8
                        )
                        plsc.addupdate_scatter(cnt_v, [rows, col], val)

            out_pending = pltpu.async_copy(
                cnt_v,
                cnt_hbm.at[
                    pl.ds(wid * rows_per_w + g * rows_per_half, rows_per_half)
                ],
                out_sem,
            )
            if g + 1 < b_groups:
                out_pending.wait()
        out_pending.wait()

    return hist


def _mm_body(cnt_ref, cb_ref, o_ref):
    packed = cnt_ref[...]
    acc = None
    for p in range(PLANES):
        plane = lax.shift_right_logical(packed, 8 * p) & 0xFF
        term = jnp.dot(
            plane.astype(jnp.bfloat16),
            cb_ref[p],
            preferred_element_type=jnp.float32,
        )
        acc = term if acc is None else acc + term
    o_ref[...] = acc


def _build_matmul(n_rows: int, d: int, block_rows: int):
    grid = (n_rows // block_rows,)
    return pl.pallas_call(
        _mm_body,
        grid=grid,
        in_specs=[
            pl.BlockSpec((block_rows, C_PAD), lambda i: (i, 0)),
            pl.BlockSpec((PLANES, C_PAD, d), lambda i: (0, 0, 0)),
        ],
        out_specs=pl.BlockSpec((block_rows, d), lambda i: (i, 0)),
        out_shape=jax.ShapeDtypeStruct((n_rows, d), jnp.float32),
    )


@functools.lru_cache(maxsize=None)
def _build(b, n, j, v, d):
    n_rows = b * n
    hist = _build_hist(b, n, j)
    matmul = _build_matmul(n_rows, d, block_rows=4096)

    def run(idx, codebook):
        idx_t = jnp.transpose(idx.astype(jnp.int32), (1, 2, 0))
        counts = hist(idx_t)
        cb_pad = jnp.pad(codebook.astype(jnp.float32), ((0, C_BINS - v), (0, 0)))
        cb_planes = jnp.pad(
            cb_pad.reshape(PLANES, C_PACK, d), ((0, 0), (0, C_PAD - C_PACK), (0, 0))
        ).astype(jnp.bfloat16)
        return matmul(counts, cb_planes).reshape(b, n, d)

    return run


def kernel(node2node_shortest_path_length_type, codebook):
    b, n, j = node2node_shortest_path_length_type.shape
    v, d = codebook.shape
    return _build(b, n, j, v, d)(node2node_shortest_path_length_type, codebook)


# speedup vs baseline: 1.1685x; 1.0074x over previous
import functools

import jax
import jax.numpy as jnp
from jax import lax
from jax.experimental import pallas as pl
from jax.experimental.pallas import tpu as pltpu
from jax.experimental.pallas import tpu_sc as plsc

NC = 2
NS = 16
NW = NC * NS
LANES = 16

C_BINS = 272
PLANES = 4
C_PACK = C_BINS // PLANES
C_PAD = 80


def _build_hist(b: int, n: int, n_idx: int):
    n_rows = b * n
    assert b % NW == 0
    b_per_w = b // NW
    rows_per_w = b_per_w * n
    assert b_per_w % LANES == 0
    b_groups = b_per_w // LANES
    SLAB = 128
    w_per_slab = SLAB // b_per_w
    i_chunk = n // w_per_slab
    mesh = plsc.VectorSubcoreMesh(core_axis_name="c", subcore_axis_name="s")

    rows_per_half = rows_per_w // b_groups
    n_ii = n // i_chunk
    n_stages = b_groups * n_ii

    @functools.partial(
        pl.kernel,
        out_type=jax.ShapeDtypeStruct((n_rows, C_PAD), jnp.int32),
        mesh=mesh,
        compiler_params=pltpu.CompilerParams(needs_layout_passes=False),
        scratch_types=[
            pltpu.VMEM((2, i_chunk, n_idx, SLAB), jnp.int32),
            pltpu.VMEM((rows_per_half, C_PAD), jnp.int32),
            pltpu.SemaphoreType.DMA,
            pltpu.SemaphoreType.DMA,
            pltpu.SemaphoreType.DMA,
        ],
    )
    def hist(idx_hbm, cnt_hbm, idx_v, cnt_v, sem0, sem1, out_sem):
        wid = lax.axis_index("s") * NC + lax.axis_index("c")
        slab = wid // w_per_slab
        quarter = wid % w_per_slab
        zeros = jnp.zeros((LANES,), jnp.int32)
        row_step = lax.iota(jnp.int32, LANES) * n
        sems = (sem0, sem1)

        def start(s):
            ii = s % n_ii
            return pltpu.async_copy(
                idx_hbm.at[
                    pl.ds(ii * i_chunk, i_chunk), :, pl.ds(slab * SLAB, SLAB)
                ],
                idx_v.at[s % 2],
                sems[s % 2],
            )

        pending = {0: start(0)}
        out_pending = None
        for g in range(b_groups):
            @plsc.parallel_loop(0, rows_per_half, unroll=8)
            def _zero(r):
                for c in range(C_PAD // LANES):
                    cnt_v[r, pl.ds(c * LANES, LANES)] = zeros

            for ii in range(n_ii):
                s = g * n_ii + ii
                if s + 1 < n_stages:
                    pending[s + 1] = start(s + 1)
                pending.pop(s).wait()
                buf = s % 2

                @plsc.parallel_loop(0, i_chunk * 4, unroll=1)
                def _scatter(t):
                    il = t // 4
                    jg = t % 4
                    rows = row_step + (ii * i_chunk + il)
                    lane0 = quarter * b_per_w + g * LANES
                    for jj in range(n_idx // 4):
                        j = jg * (n_idx // 4) + jj
                        v = idx_v[buf, il, j, pl.ds(lane0, LANES)]
                        q = (
                            (v >= C_PACK).astype(jnp.int32)
                            + (v >= 2 * C_PACK).astype(jnp.int32)
                            + (v >= 3 * C_PACK).astype(jnp.int32)
                        )
                        col = v - q * C_PACK
                        val = lax.shift_left(
                            jnp.full((LANES,), 1, jnp.int32), q * 8
                        )
                        plsc.addupdate_scatter(cnt_v, [rows, col], val)

            out_pending = pltpu.async_copy(
                cnt_v,
                cnt_hbm.at[
                    pl.ds(wid * rows_per_w + g * rows_per_half, rows_per_half)
                ],
                out_sem,
            )
            if g + 1 < b_groups:
                out_pending.wait()
        out_pending.wait()

    return hist


def _mm_body(cnt_ref, cb_ref, o_ref):
    packed = cnt_ref[...]
    acc = None
    for p in range(PLANES):
        plane = lax.shift_right_logical(packed, 8 * p) & 0xFF
        term = jnp.dot(
            plane.astype(jnp.bfloat16),
            cb_ref[p],
            preferred_element_type=jnp.float32,
        )
        acc = term if acc is None else acc + term
    o_ref[...] = acc


def _build_matmul(n_rows: int, d: int, block_rows: int):
    grid = (n_rows // block_rows,)
    return pl.pallas_call(
        _mm_body,
        grid=grid,
        in_specs=[
            pl.BlockSpec((block_rows, C_PAD), lambda i: (i, 0)),
            pl.BlockSpec((PLANES, C_PAD, d), lambda i: (0, 0, 0)),
        ],
        out_specs=pl.BlockSpec((block_rows, d), lambda i: (i, 0)),
        out_shape=jax.ShapeDtypeStruct((n_rows, d), jnp.float32),
    )


@functools.lru_cache(maxsize=None)
def _build(b, n, j, v, d):
    n_rows = b * n
    hist = _build_hist(b, n, j)
    matmul = _build_matmul(n_rows, d, block_rows=8192)

    def run(idx, codebook):
        idx_t = jnp.transpose(idx.astype(jnp.int32), (1, 2, 0))
        counts = hist(idx_t)
        cb_pad = jnp.pad(codebook.astype(jnp.float32), ((0, C_BINS - v), (0, 0)))
        cb_planes = jnp.pad(
            cb_pad.reshape(PLANES, C_PACK, d), ((0, 0), (0, C_PAD - C_PACK), (0, 0))
        ).astype(jnp.bfloat16)
        return matmul(counts, cb_planes).reshape(b, n, d)

    return run


def kernel(node2node_shortest_path_length_type, codebook):
    b, n, j = node2node_shortest_path_length_type.shape
    v, d = codebook.shape
    return _build(b, n, j, v, d)(node2node_shortest_path_length_type, codebook)
